# baseline (speedup 1.0000x reference)
import jax
import jax.numpy as jnp
from jax import lax
from jax.experimental import pallas as pl
from jax.experimental.pallas import tpu as pltpu
from jax.experimental.pallas import tpu_sc as plsc


def kernel(seq_len, emb):
    n = (emb.shape[0] + 1) // 2
    d = emb.shape[1]

    mesh = plsc.VectorSubcoreMesh(core_axis_name="c", subcore_axis_name="s")
    num_cores = mesh.num_cores
    num_workers = num_cores * mesh.num_subcores
    rows_per_w = n // num_workers
    win_rows = n + rows_per_w - 1

    @pl.kernel(
        out_type=jax.ShapeDtypeStruct((n * n * d,), emb.dtype),
        mesh=mesh,
        scratch_types=[
            pltpu.SemaphoreType.DMA,
        ],
    )
    def rel_pos_kernel(emb_hbm, out_hbm, sem):
        wid = lax.axis_index("s") * num_cores + lax.axis_index("c")
        i0 = wid * rows_per_w
        copies = []
        for k in range(rows_per_w):
            cp = pltpu.async_copy(
                emb_hbm.at[pl.ds((n - 1 - (i0 + k)) * d, n * d)],
                out_hbm.at[pl.ds((i0 + k) * n * d, n * d)],
                sem,
            )
            copies.append(cp)
        for cp in copies:
            cp.wait()

    return rel_pos_kernel(emb.reshape(-1)).reshape(n, n, d)


# speedup vs baseline: 60.5628x; 60.5628x over previous
import jax
import jax.numpy as jnp
from jax import lax
from jax.experimental import pallas as pl
from jax.experimental.pallas import tpu as pltpu
from jax.experimental.pallas import tpu_sc as plsc


def kernel(seq_len, emb):
    n = (emb.shape[0] + 1) // 2
    d = emb.shape[1]

    mesh = plsc.VectorSubcoreMesh(core_axis_name="c", subcore_axis_name="s")
    num_cores = mesh.num_cores
    num_workers = num_cores * mesh.num_subcores
    rows_per_w = n // num_workers
    win_rows = n + rows_per_w - 1

    @pl.kernel(
        out_type=jax.ShapeDtypeStruct((n * n * d,), emb.dtype),
        mesh=mesh,
        scratch_types=[
            pltpu.VMEM((win_rows * d,), emb.dtype),
            pltpu.SemaphoreType.DMA,
        ],
    )
    def rel_pos_kernel(emb_hbm, out_hbm, win_v, sem):
        wid = lax.axis_index("s") * num_cores + lax.axis_index("c")
        i0 = wid * rows_per_w
        win_start = n - rows_per_w - i0
        half = n // 2
        cut = half + rows_per_w
        pltpu.sync_copy(emb_hbm.at[pl.ds(win_start * d, cut * d)],
                        win_v.at[pl.ds(0, cut * d)])
        copies = []
        for k in range(rows_per_w):
            cp = pltpu.async_copy(
                win_v.at[pl.ds((rows_per_w - 1 - k) * d, half * d)],
                out_hbm.at[pl.ds((i0 + k) * n * d, half * d)],
                sem,
            )
            copies.append(cp)
        pltpu.sync_copy(emb_hbm.at[pl.ds((win_start + cut) * d,
                                         (win_rows - cut) * d)],
                        win_v.at[pl.ds(cut * d, (win_rows - cut) * d)])
        for k in range(rows_per_w):
            cp = pltpu.async_copy(
                win_v.at[pl.ds(((rows_per_w - 1 - k) + half) * d, half * d)],
                out_hbm.at[pl.ds(((i0 + k) * n + half) * d, half * d)],
                sem,
            )
            copies.append(cp)
        for cp in copies:
            cp.wait()

    return rel_pos_kernel(emb.reshape(-1)).reshape(n, n, d)


# speedup vs baseline: 60.6555x; 1.0015x over previous
import jax
import jax.numpy as jnp
from jax import lax
from jax.experimental import pallas as pl
from jax.experimental.pallas import tpu as pltpu
from jax.experimental.pallas import tpu_sc as plsc


def kernel(seq_len, emb):
    n = (emb.shape[0] + 1) // 2
    d = emb.shape[1]

    mesh = plsc.VectorSubcoreMesh(core_axis_name="c", subcore_axis_name="s")
    num_cores = mesh.num_cores
    num_workers = num_cores * mesh.num_subcores
    rows_per_w = n // num_workers
    win_rows = n + rows_per_w - 1

    @pl.kernel(
        out_type=jax.ShapeDtypeStruct((n * n * d,), emb.dtype),
        mesh=mesh,
        scratch_types=[
            pltpu.VMEM((win_rows * d,), emb.dtype),
            pltpu.SemaphoreType.DMA,
        ],
    )
    def rel_pos_kernel(emb_hbm, out_hbm, win_v, sem):
        wid = lax.axis_index("s") * num_cores + lax.axis_index("c")
        i0 = wid * rows_per_w
        win_start = n - rows_per_w - i0
        pltpu.sync_copy(emb_hbm.at[pl.ds(win_start * d, win_rows * d)], win_v)
        copies = []
        for k in range(rows_per_w):
            cp = pltpu.async_copy(
                win_v.at[pl.ds((rows_per_w - 1 - k) * d, n * d)],
                out_hbm.at[pl.ds((i0 + k) * n * d, n * d)],
                sem,
            )
            copies.append(cp)
        for cp in copies:
            cp.wait()

    return rel_pos_kernel(emb.reshape(-1)).reshape(n, n, d)
